# baseline (speedup 1.0000x reference)
import jax
import jax.numpy as jnp
from jax.experimental import pallas as pl
from jax.experimental.pallas import tpu as pltpu

B, L, D = 2, 2048, 768
H = 128
TAU = 0.05
OUT_DIM = D + 2 * H
S_HID = max(128, OUT_DIM // 2)

TI = 512
TJ = 512
NI = L // TI
NJ = L // TJ

_F32 = jnp.float32
_BF16 = jnp.bfloat16
_NEG_BIG = -1e30


def _lrelu(x):
    return jnp.where(x >= 0, x, 0.2 * x)


def _prep_body(x_ref, w1_ref, hn_ref, xp_ref):
    x = x_ref[0]
    nrm = jnp.sqrt(jnp.sum(x * x, axis=1, keepdims=True))
    inv = 1.0 / jnp.maximum(nrm, 1e-8)
    hn_ref[0] = (x * inv).astype(_BF16)
    xp_ref[0] = jnp.dot(x, w1_ref[...], preferred_element_type=_F32)


def _layer1_body(hn_i_ref, hn_j_ref, xp_i_ref, xp_j_ref, asrc_ref, adst_ref,
                 we_ref, ae_ref, b1_ref, w2_ref,
                 mask_ref, h1_ref, xp2_ref,
                 acc_ref, m_ref, l_ref):
    j = pl.program_id(2)

    @pl.when(j == 0)
    def _():
        acc_ref[...] = jnp.zeros_like(acc_ref)
        m_ref[...] = jnp.full_like(m_ref, _NEG_BIG)
        l_ref[...] = jnp.zeros_like(l_ref)

    hn_i = hn_i_ref[0]
    hn_j = hn_j_ref[0]
    sim = jax.lax.dot_general(hn_i, hn_j, (((1,), (1,)), ((), ())),
                              preferred_element_type=_F32)
    mask = sim > TAU
    mask_ref[0] = mask.astype(_BF16)

    xp_j = xp_j_ref[0]
    xp_i = xp_i_ref[0]
    a_s = jax.lax.dot_general(asrc_ref[...], xp_j, (((1,), (1,)), ((), ())),
                              preferred_element_type=_F32)
    a_d = jax.lax.dot_general(xp_i, adst_ref[...], (((1,), (1,)), ((), ())),
                              preferred_element_type=_F32)
    c = jnp.sum(we_ref[...] * ae_ref[...])
    e = _lrelu(a_s + a_d + c)
    e = jnp.where(mask, e, -jnp.inf)

    m_prev = m_ref[...]
    m_new = jnp.maximum(m_prev, jnp.max(e, axis=1, keepdims=True))
    m_new = jnp.maximum(m_new, _NEG_BIG)
    scale = jnp.exp(m_prev - m_new)
    p = jnp.exp(e - m_new)
    l_ref[...] = l_ref[...] * scale + jnp.sum(p, axis=1, keepdims=True)
    acc_ref[...] = acc_ref[...] * scale + jnp.dot(
        p, xp_j, preferred_element_type=_F32)
    m_ref[...] = m_new

    @pl.when(j == NJ - 1)
    def _():
        h1 = acc_ref[...] / l_ref[...] + b1_ref[...]
        h1 = jnp.maximum(h1, 0.0)
        h1_ref[0] = h1
        xp2_ref[0] = jnp.dot(h1, w2_ref[...], preferred_element_type=_F32)


def _layer2_body(mask_ref, x_i_ref, h1_i_ref, xp_i_ref, xp_j_ref,
                 asrc_ref, adst_ref, we_ref, ae_ref, b2_ref,
                 s1x_ref, s1h1_ref, s1h2_ref, s1b_ref, s2w_ref,
                 out_ref,
                 acc_ref, m_ref, l_ref, gm_ref, gl_ref, gx_ref, g1_ref, g2_ref):
    i = pl.program_id(1)
    j = pl.program_id(2)

    @pl.when(j == 0)
    def _():
        acc_ref[...] = jnp.zeros_like(acc_ref)
        m_ref[...] = jnp.full_like(m_ref, _NEG_BIG)
        l_ref[...] = jnp.zeros_like(l_ref)

    @pl.when((i == 0) & (j == 0))
    def _():
        gm_ref[...] = jnp.full_like(gm_ref, _NEG_BIG)
        gl_ref[...] = jnp.zeros_like(gl_ref)
        gx_ref[...] = jnp.zeros_like(gx_ref)
        g1_ref[...] = jnp.zeros_like(g1_ref)
        g2_ref[...] = jnp.zeros_like(g2_ref)

    mask = mask_ref[0] > 0.5
    xp_j = xp_j_ref[0]
    xp_i = xp_i_ref[0]
    a_s = jax.lax.dot_general(asrc_ref[...], xp_j, (((1,), (1,)), ((), ())),
                              preferred_element_type=_F32)
    a_d = jax.lax.dot_general(xp_i, adst_ref[...], (((1,), (1,)), ((), ())),
                              preferred_element_type=_F32)
    c = jnp.sum(we_ref[...] * ae_ref[...])
    e = _lrelu(a_s + a_d + c)
    e = jnp.where(mask, e, -jnp.inf)

    m_prev = m_ref[...]
    m_new = jnp.maximum(m_prev, jnp.max(e, axis=1, keepdims=True))
    m_new = jnp.maximum(m_new, _NEG_BIG)
    scale = jnp.exp(m_prev - m_new)
    p = jnp.exp(e - m_new)
    l_ref[...] = l_ref[...] * scale + jnp.sum(p, axis=1, keepdims=True)
    acc_ref[...] = acc_ref[...] * scale + jnp.dot(
        p, xp_j, preferred_element_type=_F32)
    m_ref[...] = m_new

    @pl.when(j == NJ - 1)
    def _():
        h2 = acc_ref[...] / l_ref[...] + b2_ref[...]
        h2 = jnp.maximum(h2, 0.0)
        x_i = x_i_ref[0]
        h1_i = h1_i_ref[0]
        t = jnp.dot(x_i, s1x_ref[...], preferred_element_type=_F32)
        t = t + jnp.dot(h1_i, s1h1_ref[...], preferred_element_type=_F32)
        t = t + jnp.dot(h2, s1h2_ref[...], preferred_element_type=_F32)
        t = jnp.tanh(t + s1b_ref[...])
        s = jax.lax.dot_general(t, s2w_ref[...], (((1,), (1,)), ((), ())),
                                preferred_element_type=_F32)
        gm_prev = gm_ref[...]
        gm_new = jnp.maximum(gm_prev, jnp.max(s, axis=(0, 1), keepdims=True))
        sc = jnp.exp(gm_prev - gm_new)
        w = jnp.exp(s - gm_new)
        gl_ref[...] = gl_ref[...] * sc + jnp.sum(w, axis=(0, 1), keepdims=True)
        gx_ref[...] = gx_ref[...] * sc + jax.lax.dot_general(
            w, x_i, (((0,), (0,)), ((), ())), preferred_element_type=_F32)
        g1_ref[...] = g1_ref[...] * sc + jax.lax.dot_general(
            w, h1_i, (((0,), (0,)), ((), ())), preferred_element_type=_F32)
        g2_ref[...] = g2_ref[...] * sc + jax.lax.dot_general(
            w, h2, (((0,), (0,)), ((), ())), preferred_element_type=_F32)
        gm_ref[...] = gm_new

        @pl.when(i == NI - 1)
        def _():
            gl = gl_ref[...]
            out_ref[0, :, 0:D] = gx_ref[...] / gl
            out_ref[0, :, D:D + H] = g1_ref[...] / gl
            out_ref[0, :, D + H:OUT_DIM] = g2_ref[...] / gl


def kernel(hidden, attention_mask, W1, att_src1, att_dst1, We1, att_edge1, b1,
           W2, att_src2, att_dst2, We2, att_edge2, b2, S1_w, S1_b, S2_w, S2_b):
    del attention_mask, S2_b
    x = hidden

    hn, xp1 = pl.pallas_call(
        _prep_body,
        grid=(B, NI),
        in_specs=[
            pl.BlockSpec((1, TI, D), lambda b, i: (b, i, 0)),
            pl.BlockSpec((D, H), lambda b, i: (0, 0)),
        ],
        out_specs=[
            pl.BlockSpec((1, TI, D), lambda b, i: (b, i, 0)),
            pl.BlockSpec((1, TI, H), lambda b, i: (b, i, 0)),
        ],
        out_shape=[
            jax.ShapeDtypeStruct((B, L, D), _BF16),
            jax.ShapeDtypeStruct((B, L, H), _F32),
        ],
    )(x, W1)

    row = lambda v: v.reshape(1, -1)

    mask, h1, xp2 = pl.pallas_call(
        _layer1_body,
        grid=(B, NI, NJ),
        in_specs=[
            pl.BlockSpec((1, TI, D), lambda b, i, j: (b, i, 0)),
            pl.BlockSpec((1, TJ, D), lambda b, i, j: (b, j, 0)),
            pl.BlockSpec((1, TI, H), lambda b, i, j: (b, i, 0)),
            pl.BlockSpec((1, TJ, H), lambda b, i, j: (b, j, 0)),
            pl.BlockSpec((1, H), lambda b, i, j: (0, 0)),
            pl.BlockSpec((1, H), lambda b, i, j: (0, 0)),
            pl.BlockSpec((1, H), lambda b, i, j: (0, 0)),
            pl.BlockSpec((1, H), lambda b, i, j: (0, 0)),
            pl.BlockSpec((1, H), lambda b, i, j: (0, 0)),
            pl.BlockSpec((H, H), lambda b, i, j: (0, 0)),
        ],
        out_specs=[
            pl.BlockSpec((1, TI, TJ), lambda b, i, j: (b, i, j)),
            pl.BlockSpec((1, TI, H), lambda b, i, j: (b, i, 0)),
            pl.BlockSpec((1, TI, H), lambda b, i, j: (b, i, 0)),
        ],
        out_shape=[
            jax.ShapeDtypeStruct((B, L, L), _BF16),
            jax.ShapeDtypeStruct((B, L, H), _F32),
            jax.ShapeDtypeStruct((B, L, H), _F32),
        ],
        scratch_shapes=[
            pltpu.VMEM((TI, H), _F32),
            pltpu.VMEM((TI, 1), _F32),
            pltpu.VMEM((TI, 1), _F32),
        ],
    )(hn, hn, xp1, xp1, row(att_src1), row(att_dst1), row(We1),
      row(att_edge1), row(b1), W2)

    pooled = pl.pallas_call(
        _layer2_body,
        grid=(B, NI, NJ),
        in_specs=[
            pl.BlockSpec((1, TI, TJ), lambda b, i, j: (b, i, j)),
            pl.BlockSpec((1, TI, D), lambda b, i, j: (b, i, 0)),
            pl.BlockSpec((1, TI, H), lambda b, i, j: (b, i, 0)),
            pl.BlockSpec((1, TI, H), lambda b, i, j: (b, i, 0)),
            pl.BlockSpec((1, TJ, H), lambda b, i, j: (b, j, 0)),
            pl.BlockSpec((1, H), lambda b, i, j: (0, 0)),
            pl.BlockSpec((1, H), lambda b, i, j: (0, 0)),
            pl.BlockSpec((1, H), lambda b, i, j: (0, 0)),
            pl.BlockSpec((1, H), lambda b, i, j: (0, 0)),
            pl.BlockSpec((1, H), lambda b, i, j: (0, 0)),
            pl.BlockSpec((D, S_HID), lambda b, i, j: (0, 0)),
            pl.BlockSpec((H, S_HID), lambda b, i, j: (0, 0)),
            pl.BlockSpec((H, S_HID), lambda b, i, j: (0, 0)),
            pl.BlockSpec((1, S_HID), lambda b, i, j: (0, 0)),
            pl.BlockSpec((1, S_HID), lambda b, i, j: (0, 0)),
        ],
        out_specs=pl.BlockSpec((1, 1, OUT_DIM), lambda b, i, j: (b, 0, 0)),
        out_shape=jax.ShapeDtypeStruct((B, 1, OUT_DIM), _F32),
        scratch_shapes=[
            pltpu.VMEM((TI, H), _F32),
            pltpu.VMEM((TI, 1), _F32),
            pltpu.VMEM((TI, 1), _F32),
            pltpu.VMEM((1, 1), _F32),
            pltpu.VMEM((1, 1), _F32),
            pltpu.VMEM((1, D), _F32),
            pltpu.VMEM((1, H), _F32),
            pltpu.VMEM((1, H), _F32),
        ],
    )(mask, x, h1, xp2, xp2, row(att_src2), row(att_dst2), row(We2),
      row(att_edge2), row(b2), S1_w[0:D, :], S1_w[D:D + H, :],
      S1_w[D + H:OUT_DIM, :], row(S1_b), S2_w.reshape(1, S_HID))

    return pooled.reshape(B, OUT_DIM)
